# baseline (speedup 1.0000x reference)
import functools

import jax
import jax.numpy as jnp
from jax import lax
from jax.experimental import pallas as pl
from jax.experimental.pallas import tpu as pltpu
from jax.experimental.pallas import tpu_sc as plsc

_NC, _NS = 2, 16
_NW = _NC * _NS
_K = 32768
_ROWS_TOTAL = 1024
_ROWS_PER_W = _ROWS_TOTAL // _NW
_SEG = _K // _NS
_L = 16


@functools.partial(
    pl.kernel,
    mesh=plsc.VectorSubcoreMesh(core_axis_name="c", subcore_axis_name="s"),
    out_type=jax.ShapeDtypeStruct((_ROWS_TOTAL, _K), jnp.float32),
    scratch_types=[
        pltpu.VMEM((_SEG,), jnp.float32),
        pltpu.VMEM((_K,), jnp.float32),
        pltpu.VMEM_SHARED((_K,), jnp.float32),
        pltpu.SemaphoreType.DMA,
    ],
)
def _sc_broadcast(mask_hbm, out_hbm, chunk_v, full_v, shared, sem):
    c = lax.axis_index("c")
    s = lax.axis_index("s")
    seg_base = s * _SEG
    pltpu.sync_copy(mask_hbm.at[pl.ds(seg_base, _SEG)], chunk_v)

    def body(i, carry):
        v = chunk_v[pl.ds(i * _L, _L)]
        chunk_v[pl.ds(i * _L, _L)] = 1.0 / (1.0 + jnp.exp(-v))
        return carry

    lax.fori_loop(0, _SEG // _L, body, 0)

    pltpu.sync_copy(chunk_v, shared.at[pl.ds(seg_base, _SEG)])
    plsc.subcore_barrier()
    pltpu.sync_copy(shared, full_v)

    base = (c * _NS + s) * _ROWS_PER_W
    copies = [
        pltpu.async_copy(full_v, out_hbm.at[base + r], sem)
        for r in range(_ROWS_PER_W)
    ]
    for cp in copies:
        cp.wait()


def kernel(x, mask):
    out = _sc_broadcast(mask.reshape(_K))
    return out.reshape(x.shape)


# speedup vs baseline: 1.5277x; 1.5277x over previous
import jax
import jax.numpy as jnp
from jax.experimental import pallas as pl
from jax.experimental.pallas import tpu as pltpu

_K = 32768
_ROWS_TOTAL = 1024
_CHUNK = 32
_NCOPY = _ROWS_TOTAL // _CHUNK


def _body(mask_ref, out_ref, buf, sem):
    s = jax.nn.sigmoid(mask_ref[...])
    buf[...] = jnp.broadcast_to(s, buf.shape)
    for i in range(_NCOPY):
        pltpu.make_async_copy(
            buf, out_ref.at[pl.ds(i * _CHUNK, _CHUNK)], sem
        ).start()
    for i in range(_NCOPY):
        pltpu.make_async_copy(
            buf, out_ref.at[pl.ds(i * _CHUNK, _CHUNK)], sem
        ).wait()


def kernel(x, mask):
    b, h, k = x.shape
    out = pl.pallas_call(
        _body,
        grid=(1,),
        in_specs=[pl.BlockSpec((1, k), lambda i: (0, 0))],
        out_specs=pl.BlockSpec(memory_space=pl.ANY),
        out_shape=jax.ShapeDtypeStruct((b * h, k), x.dtype),
        scratch_shapes=[
            pltpu.VMEM((_CHUNK, _K), jnp.float32),
            pltpu.SemaphoreType.DMA,
        ],
    )(mask.reshape(1, k))
    return out.reshape(b, h, k)


# speedup vs baseline: 1.5805x; 1.0345x over previous
import jax
import jax.numpy as jnp
from jax.experimental import pallas as pl
from jax.experimental.pallas import tpu as pltpu

_ROWS = 32


def _body(mask_ref, out_ref):
    s = jax.nn.sigmoid(mask_ref[...])
    out_ref[...] = jnp.broadcast_to(s, out_ref.shape)


def kernel(x, mask):
    b, h, k = x.shape
    rows = b * h
    out = pl.pallas_call(
        _body,
        grid=(rows // _ROWS,),
        in_specs=[pl.BlockSpec((1, k), lambda i: (0, 0))],
        out_specs=pl.BlockSpec((_ROWS, k), lambda i: (i, 0)),
        out_shape=jax.ShapeDtypeStruct((rows, k), x.dtype),
        compiler_params=pltpu.CompilerParams(
            dimension_semantics=("arbitrary",)
        ),
    )(mask.reshape(1, k))
    return out.reshape(b, h, k)
